# baseline (speedup 1.0000x reference)
import math

import jax
import jax.numpy as jnp
import numpy as np
from jax.experimental import pallas as pl
from jax.experimental.pallas import tpu as pltpu

_LQ = 2048
_LK = 2048
_FACTOR = 5
_U = _FACTOR * int(np.ceil(np.log(_LQ)))
_U_PART = _FACTOR * int(np.ceil(np.log(_LK)))
_UP = 64
_ROW_TILE = 256

_NEG_BIG = np.float32(-1e30)
_NEG_HUGE = np.float32(-3e38)


def _rotl32(x, d):
    return ((x << np.uint32(d)) | (x >> np.uint32(32 - d))).astype(np.uint32)


def _threefry2x32(k0, k1, x0, x1):
    x0 = x0.astype(np.uint32).copy()
    x1 = x1.astype(np.uint32).copy()
    ks = [np.uint32(k0), np.uint32(k1),
          np.uint32(np.uint32(k0) ^ np.uint32(k1) ^ np.uint32(0x1BD11BDA))]
    r1 = (13, 15, 26, 6)
    r2 = (17, 29, 16, 24)
    with np.errstate(over='ignore'):
        x0 += ks[0]
        x1 += ks[1]
        for i, rots in enumerate((r1, r2, r1, r2, r1)):
            for r in rots:
                x0 += x1
                x1 = _rotl32(x1, r)
                x1 ^= x0
            x0 += ks[(i + 1) % 3]
            x1 += ks[(i + 2) % 3] + np.uint32(i + 1)
    return x0, x1


def _np_randint_key42(shape, span):
    n = int(np.prod(shape))
    b1, b2 = _threefry2x32(np.uint32(0), np.uint32(42),
                           np.zeros(2, np.uint32),
                           np.arange(2, dtype=np.uint32))
    k2 = (b1[1], b2[1])
    o1, o2 = _threefry2x32(k2[0], k2[1],
                           np.zeros(n, np.uint32),
                           np.arange(n, dtype=np.uint32))
    return ((o1 ^ o2) % np.uint32(span)).astype(np.int32).reshape(shape)


_INDEX_SAMPLE = _np_randint_key42((_LQ, _U_PART), _LK)
_COUNTS = np.zeros((_LQ, _LK), np.float32)
np.add.at(_COUNTS, (np.arange(_LQ)[:, None], _INDEX_SAMPLE), 1.0)
_COUNTS_T = np.ascontiguousarray(_COUNTS.T)
_MASK_T = np.where(_COUNTS_T > 0, np.float32(0.0), _NEG_BIG)


def _prob_attn_kernel(nheads, ct_hbm, mk_hbm, q_hbm, k_hbm, v_hbm, out_ref,
                      ct_ref, mk_ref, q_scr, k_scr, v_scr, ct_sem, mk_sem,
                      qkv_sem):
    i = pl.program_id(0)
    nsteps = pl.num_programs(0)

    def slab_copies(step, slot):
        b = step // nheads
        h = step % nheads
        return [
            pltpu.make_async_copy(t.at[b, :, h, :], s.at[slot],
                                  qkv_sem.at[ti, slot])
            for ti, (t, s) in enumerate(
                ((q_hbm, q_scr), (k_hbm, k_scr), (v_hbm, v_scr)))
        ]

    @pl.when(i == 0)
    def _():
        cp = pltpu.make_async_copy(ct_hbm, ct_ref, ct_sem)
        cp.start()
        mp = pltpu.make_async_copy(mk_hbm, mk_ref, mk_sem)
        mp.start()
        for c in slab_copies(0, 0):
            c.start()
        cp.wait()
        mp.wait()

    slot = jax.lax.rem(i, 2)
    for c in slab_copies(i, slot):
        c.wait()

    @pl.when(i + 1 < nsteps)
    def _():
        for c in slab_copies(i + 1, jax.lax.rem(i + 1, 2)):
            c.start()

    q = q_scr[slot]
    k = k_scr[slot]
    v = v_scr[slot]

    nt = _LQ // _ROW_TILE
    m_tiles = []
    for rb in range(nt):
        qt = q[rb * _ROW_TILE:(rb + 1) * _ROW_TILE, :]
        s_t = jax.lax.dot_general(
            k, qt, (((1,), (1,)), ((), ())),
            preferred_element_type=jnp.float32)
        c_t = ct_ref[:, rb * _ROW_TILE:(rb + 1) * _ROW_TILE]
        masked = s_t + mk_ref[:, rb * _ROW_TILE:(rb + 1) * _ROW_TILE]
        mx = jnp.max(masked, axis=0, keepdims=True)
        sm = jnp.sum(s_t * c_t, axis=0, keepdims=True)
        m_tiles.append(mx - sm / _LK)
    m_rows = jnp.concatenate(m_tiles, axis=1)

    jcol = jax.lax.broadcasted_iota(
        jnp.int32, (_UP, 1), 0).astype(jnp.float32)
    ti0 = jax.lax.broadcasted_iota(jnp.int32, (_ROW_TILE, _ROW_TILE), 0)
    ti1 = jax.lax.broadcasted_iota(jnp.int32, (_ROW_TILE, _ROW_TILE), 1)
    tril = (ti0 < ti1).astype(jnp.float32)
    ident = (ti0 == ti1).astype(jnp.float32)

    def split3(x):
        hi = x.astype(jnp.bfloat16).astype(jnp.float32)
        r = x - hi
        mid = r.astype(jnp.bfloat16).astype(jnp.float32)
        return hi, mid, r - mid

    def dot_t(a, b):
        return jax.lax.dot_general(a, b, (((1,), (1,)), ((), ())),
                                   preferred_element_type=jnp.float32)

    def exact_dot_t(a, b_parts):
        out = dot_t(a, b_parts[0])
        for p in b_parts[1:]:
            out = out + dot_t(a, p)
        return out

    m_parts = split3(m_rows)
    m_col = jnp.concatenate(
        [exact_dot_t(ident,
                     tuple(p[:, rb * _ROW_TILE:(rb + 1) * _ROW_TILE]
                           for p in m_parts))
         for rb in range(nt)], axis=0)
    run = jnp.zeros((1, 1), jnp.float32)
    ohc_tiles = []
    for rb in range(nt):
        m_row = m_rows[:, rb * _ROW_TILE:(rb + 1) * _ROW_TILE]
        gtc = jnp.sum(jnp.where(m_col > m_row, 1.0, 0.0),
                      axis=0, keepdims=True)
        cand = jnp.where(gtc < float(_U), 1.0, 0.0)
        local = jax.lax.dot_general(
            cand, tril, (((1,), (0,)), ((), ())),
            preferred_element_type=jnp.float32)
        slot = local + run
        run = run + jnp.sum(cand, keepdims=True)
        ohc_tiles.append(
            jnp.where((slot == jcol) & (cand > 0.0), 1.0, 0.0))
    ohc = jnp.concatenate(ohc_tiles, axis=1)

    giota_int = jax.lax.broadcasted_iota(jnp.int32, (1, _LQ), 1)
    gihi = (giota_int // 256).astype(jnp.float32)
    gilo = (giota_int % 256).astype(jnp.float32)
    mc_col = exact_dot_t(ohc, m_parts)
    mc_row = sum(dot_t(p, ohc) for p in m_parts)
    gi_col = 256.0 * dot_t(ohc, gihi) + dot_t(ohc, gilo)
    gi_row = 256.0 * dot_t(gihi, ohc) + dot_t(gilo, ohc)
    ones_row = jnp.ones((1, _LQ), jnp.float32)
    occ_col = dot_t(ohc, ones_row)
    occ_row = dot_t(ones_row, ohc)
    mc_col = jnp.where(occ_col > 0.0, mc_col, _NEG_HUGE)
    mc_row = jnp.where(occ_row > 0.0, mc_row, _NEG_HUGE)
    gi_col = jnp.where(occ_col > 0.0, gi_col, float(_LQ) + jcol)
    srow = jax.lax.broadcasted_iota(
        jnp.int32, (1, _UP), 1).astype(jnp.float32)
    gi_row = jnp.where(occ_row > 0.0, gi_row, float(_LQ) + srow)
    beats = ((mc_col > mc_row) |
             ((mc_col == mc_row) & (gi_col < gi_row)))
    rr = jnp.sum(jnp.where(beats, 1.0, 0.0),
                 axis=0, keepdims=True)
    perm = jnp.where((rr == jcol) & (jcol < float(_U)),
                     1.0, 0.0)
    oh = jax.lax.dot_general(
        perm, ohc, (((1,), (0,)), ((), ())),
        preferred_element_type=jnp.float32)

    q_red = jax.lax.dot_general(
        oh, q, (((1,), (0,)), ((), ())),
        preferred_element_type=jnp.float32)

    scores = jax.lax.dot_general(
        q_red, k, (((1,), (1,)), ((), ())),
        preferred_element_type=jnp.float32)
    scores = scores * np.float32(1.0 / math.sqrt(64))
    smax = jnp.max(scores, axis=-1, keepdims=True)
    e = jnp.exp(scores - smax)
    attn = e / jnp.sum(e, axis=-1, keepdims=True)
    update = jax.lax.dot_general(
        attn, v, (((1,), (0,)), ((), ())),
        preferred_element_type=jnp.float32)

    v_mean = jnp.sum(v, axis=0, keepdims=True) / _LK
    out_attn = jax.lax.dot_general(
        oh, update, (((0,), (0,)), ((), ())),
        preferred_element_type=jnp.float32)
    ones = jnp.ones((_UP, 1), jnp.float32)
    cov = jax.lax.dot_general(
        oh, ones, (((0,), (0,)), ((), ())),
        preferred_element_type=jnp.float32)
    out_ref[0, 0, :, :] = out_attn + (1.0 - cov) * v_mean


def kernel(queries, keys, values, attn_mask):
    del attn_mask
    import functools
    B, Lq, H, D = queries.shape
    counts_t = jnp.asarray(_COUNTS_T)
    mask_t = jnp.asarray(_MASK_T)

    grid = (B * H,)
    c_spec = pl.BlockSpec(memory_space=pl.ANY)
    out_spec = pl.BlockSpec((1, 1, Lq, D), lambda i: (i // H, i % H, 0, 0))

    return pl.pallas_call(
        functools.partial(_prob_attn_kernel, H),
        grid=grid,
        in_specs=[c_spec, c_spec, c_spec, c_spec, c_spec],
        out_specs=out_spec,
        out_shape=jax.ShapeDtypeStruct((B, H, Lq, D), jnp.float32),
        scratch_shapes=[
            pltpu.VMEM((_LK, _LQ), jnp.float32),
            pltpu.VMEM((_LK, _LQ), jnp.float32),
            pltpu.VMEM((2, _LQ, 64), jnp.float32),
            pltpu.VMEM((2, _LK, 64), jnp.float32),
            pltpu.VMEM((2, _LK, 64), jnp.float32),
            pltpu.SemaphoreType.DMA,
            pltpu.SemaphoreType.DMA,
            pltpu.SemaphoreType.DMA((3, 2)),
        ],
    )(counts_t, mask_t, queries, keys, values)


# speedup vs baseline: 1.1030x; 1.1030x over previous
import math

import jax
import jax.numpy as jnp
import numpy as np
from jax.experimental import pallas as pl
from jax.experimental.pallas import tpu as pltpu

_LQ = 2048
_LK = 2048
_FACTOR = 5
_U = _FACTOR * int(np.ceil(np.log(_LQ)))
_U_PART = _FACTOR * int(np.ceil(np.log(_LK)))
_UP = 64
_ROW_TILE = 256

_NEG_BIG = np.float32(-1e30)
_NEG_HUGE = np.float32(-3e38)


def _rotl32(x, d):
    return ((x << np.uint32(d)) | (x >> np.uint32(32 - d))).astype(np.uint32)


def _threefry2x32(k0, k1, x0, x1):
    x0 = x0.astype(np.uint32).copy()
    x1 = x1.astype(np.uint32).copy()
    ks = [np.uint32(k0), np.uint32(k1),
          np.uint32(np.uint32(k0) ^ np.uint32(k1) ^ np.uint32(0x1BD11BDA))]
    r1 = (13, 15, 26, 6)
    r2 = (17, 29, 16, 24)
    with np.errstate(over='ignore'):
        x0 += ks[0]
        x1 += ks[1]
        for i, rots in enumerate((r1, r2, r1, r2, r1)):
            for r in rots:
                x0 += x1
                x1 = _rotl32(x1, r)
                x1 ^= x0
            x0 += ks[(i + 1) % 3]
            x1 += ks[(i + 2) % 3] + np.uint32(i + 1)
    return x0, x1


def _np_randint_key42(shape, span):
    n = int(np.prod(shape))
    b1, b2 = _threefry2x32(np.uint32(0), np.uint32(42),
                           np.zeros(2, np.uint32),
                           np.arange(2, dtype=np.uint32))
    k2 = (b1[1], b2[1])
    o1, o2 = _threefry2x32(k2[0], k2[1],
                           np.zeros(n, np.uint32),
                           np.arange(n, dtype=np.uint32))
    return ((o1 ^ o2) % np.uint32(span)).astype(np.int32).reshape(shape)


_INDEX_SAMPLE = _np_randint_key42((_LQ, _U_PART), _LK)
_COUNTS = np.zeros((_LQ, _LK), np.float32)
np.add.at(_COUNTS, (np.arange(_LQ)[:, None], _INDEX_SAMPLE), 1.0)
_COUNTS_T = np.ascontiguousarray(_COUNTS.T)
_MASK_T = np.where(_COUNTS_T > 0, np.float32(0.0), _NEG_BIG)


def _prob_attn_kernel(ct_hbm, mk_hbm, q_ref, k_ref, v_ref, out_ref, ct_ref,
                      mk_ref, ct_sem, mk_sem):
    @pl.when(pl.program_id(0) == 0)
    def _():
        cp = pltpu.make_async_copy(ct_hbm, ct_ref, ct_sem)
        cp.start()
        mp = pltpu.make_async_copy(mk_hbm, mk_ref, mk_sem)
        mp.start()
        cp.wait()
        mp.wait()

    q = q_ref[0, :, :]
    k = k_ref[0, :, :]
    v = v_ref[0, :, :]

    nt = _LQ // _ROW_TILE
    m_tiles = []
    for rb in range(nt):
        qt = q[rb * _ROW_TILE:(rb + 1) * _ROW_TILE, :]
        s_t = jax.lax.dot_general(
            k, qt, (((1,), (1,)), ((), ())),
            preferred_element_type=jnp.float32)
        c_t = ct_ref[:, rb * _ROW_TILE:(rb + 1) * _ROW_TILE]
        masked = s_t + mk_ref[:, rb * _ROW_TILE:(rb + 1) * _ROW_TILE]
        mx = jnp.max(masked, axis=0, keepdims=True)
        sm = jnp.sum(s_t * c_t, axis=0, keepdims=True)
        m_tiles.append(mx - sm / _LK)
    m_rows = jnp.concatenate(m_tiles, axis=1)

    jcol = jax.lax.broadcasted_iota(
        jnp.int32, (_UP, 1), 0).astype(jnp.float32)
    ti0 = jax.lax.broadcasted_iota(jnp.int32, (_ROW_TILE, _ROW_TILE), 0)
    ti1 = jax.lax.broadcasted_iota(jnp.int32, (_ROW_TILE, _ROW_TILE), 1)
    tril = (ti0 < ti1).astype(jnp.float32)
    ident = (ti0 == ti1).astype(jnp.float32)

    def split3(x):
        hi = x.astype(jnp.bfloat16).astype(jnp.float32)
        r = x - hi
        mid = r.astype(jnp.bfloat16).astype(jnp.float32)
        return hi, mid, r - mid

    def dot_t(a, b):
        return jax.lax.dot_general(a, b, (((1,), (1,)), ((), ())),
                                   preferred_element_type=jnp.float32)

    def exact_dot_t(a, b_parts):
        out = dot_t(a, b_parts[0])
        for p in b_parts[1:]:
            out = out + dot_t(a, p)
        return out

    m_parts = split3(m_rows)
    m_col = jnp.concatenate(
        [exact_dot_t(ident,
                     tuple(p[:, rb * _ROW_TILE:(rb + 1) * _ROW_TILE]
                           for p in m_parts))
         for rb in range(nt)], axis=0)
    run = jnp.zeros((1, 1), jnp.float32)
    ohc_tiles = []
    for rb in range(nt):
        m_row = m_rows[:, rb * _ROW_TILE:(rb + 1) * _ROW_TILE]
        gtc = jnp.sum(jnp.where(m_col > m_row, 1.0, 0.0),
                      axis=0, keepdims=True)
        cand = jnp.where(gtc < float(_U), 1.0, 0.0)
        local = jax.lax.dot_general(
            cand, tril, (((1,), (0,)), ((), ())),
            preferred_element_type=jnp.float32)
        slot = local + run
        run = run + jnp.sum(cand, keepdims=True)
        ohc_tiles.append(
            jnp.where((slot == jcol) & (cand > 0.0), 1.0, 0.0))
    ohc = jnp.concatenate(ohc_tiles, axis=1)

    giota_int = jax.lax.broadcasted_iota(jnp.int32, (1, _LQ), 1)
    gihi = (giota_int // 256).astype(jnp.float32)
    gilo = (giota_int % 256).astype(jnp.float32)
    mc_col = exact_dot_t(ohc, m_parts)
    mc_row = sum(dot_t(p, ohc) for p in m_parts)
    gi_col = 256.0 * dot_t(ohc, gihi) + dot_t(ohc, gilo)
    gi_row = 256.0 * dot_t(gihi, ohc) + dot_t(gilo, ohc)
    ones_row = jnp.ones((1, _LQ), jnp.float32)
    occ_col = dot_t(ohc, ones_row)
    occ_row = dot_t(ones_row, ohc)
    mc_col = jnp.where(occ_col > 0.0, mc_col, _NEG_HUGE)
    mc_row = jnp.where(occ_row > 0.0, mc_row, _NEG_HUGE)
    gi_col = jnp.where(occ_col > 0.0, gi_col, float(_LQ) + jcol)
    srow = jax.lax.broadcasted_iota(
        jnp.int32, (1, _UP), 1).astype(jnp.float32)
    gi_row = jnp.where(occ_row > 0.0, gi_row, float(_LQ) + srow)
    beats = ((mc_col > mc_row) |
             ((mc_col == mc_row) & (gi_col < gi_row)))
    rr = jnp.sum(jnp.where(beats, 1.0, 0.0),
                 axis=0, keepdims=True)
    perm = jnp.where((rr == jcol) & (jcol < float(_U)),
                     1.0, 0.0)
    oh = jax.lax.dot_general(
        perm, ohc, (((1,), (0,)), ((), ())),
        preferred_element_type=jnp.float32)

    q_red = jax.lax.dot_general(
        oh, q, (((1,), (0,)), ((), ())),
        preferred_element_type=jnp.float32)

    scores = jax.lax.dot_general(
        q_red, k, (((1,), (1,)), ((), ())),
        preferred_element_type=jnp.float32)
    scores = scores * np.float32(1.0 / math.sqrt(64))
    smax = jnp.max(scores, axis=-1, keepdims=True)
    e = jnp.exp(scores - smax)
    attn = e / jnp.sum(e, axis=-1, keepdims=True)
    update = jax.lax.dot_general(
        attn, v, (((1,), (0,)), ((), ())),
        preferred_element_type=jnp.float32)

    v_mean = jnp.sum(v, axis=0, keepdims=True) / _LK
    out_attn = jax.lax.dot_general(
        oh, update, (((0,), (0,)), ((), ())),
        preferred_element_type=jnp.float32)
    ones = jnp.ones((_UP, 1), jnp.float32)
    cov = jax.lax.dot_general(
        oh, ones, (((0,), (0,)), ((), ())),
        preferred_element_type=jnp.float32)
    out_ref[0, 0, :, :] = out_attn + (1.0 - cov) * v_mean


def kernel(queries, keys, values, attn_mask):
    del attn_mask
    B, Lq, H, D = queries.shape
    counts_t = jnp.asarray(_COUNTS_T)
    mask_t = jnp.asarray(_MASK_T)

    q_t = jnp.transpose(queries, (0, 2, 1, 3)).reshape(B * H, Lq, D)
    k_t = jnp.transpose(keys, (0, 2, 1, 3)).reshape(B * H, Lq, D)
    v_t = jnp.transpose(values, (0, 2, 1, 3)).reshape(B * H, Lq, D)

    grid = (B * H,)
    qkv_spec = pl.BlockSpec((1, Lq, D), lambda i: (i, 0, 0))
    c_spec = pl.BlockSpec(memory_space=pl.ANY)
    out_spec = pl.BlockSpec((1, 1, Lq, D), lambda i: (i // H, i % H, 0, 0))

    return pl.pallas_call(
        _prob_attn_kernel,
        grid=grid,
        in_specs=[c_spec, c_spec, qkv_spec, qkv_spec, qkv_spec],
        out_specs=out_spec,
        out_shape=jax.ShapeDtypeStruct((B, H, Lq, D), jnp.float32),
        scratch_shapes=[
            pltpu.VMEM((_LK, _LQ), jnp.float32),
            pltpu.VMEM((_LK, _LQ), jnp.float32),
            pltpu.SemaphoreType.DMA,
            pltpu.SemaphoreType.DMA,
        ],
    )(counts_t, mask_t, q_t, k_t, v_t)
